# Initial kernel scaffold; baseline (speedup 1.0000x reference)
import jax
import jax.numpy as jnp
from jax.experimental import pallas as pl


def kernel(cell_subs, drug_subs, batch, drug_cell_batch, W_cell, b_cell, W_gnn, b_gnn, weight, W1, b1, W_mu, b_mu, W_lv, b_lv, Wq, Wk, Wv, Wo, Wp1, bp1, Wp2, bp2):
    raise NotImplementedError("write your pallas kernel here")


# speedup vs baseline: 1.2517x; 1.2517x over previous
import jax
import jax.numpy as jnp
from jax.experimental import pallas as pl
from jax.experimental.pallas import tpu as pltpu

_B = 256
_ND = 64
_NC = 8
_N = _ND + _NC
_G = 8
_HEADS = 4
_DH = 32


def _body(cs_ref, ds_ref, Wc_ref, bc_ref, Wg_ref, bg_ref, wgt_ref,
          W1_ref, b1_ref, Wmu_ref, bmu_ref, Wlv_ref, blv_ref,
          Wq_ref, Wk_ref, Wv_ref, Wo_ref, Wp1_ref, bp1_ref, Wp2_ref, bp2_ref,
          mu_ref, lv_ref, pred_ref):
    G = _G
    relu = lambda x: jnp.maximum(x, 0.0)
    bf16 = jnp.bfloat16
    f32 = jnp.float32

    def bdot(a, b, dn=None):
        a16, b16 = a.astype(bf16), b.astype(bf16)
        if dn is None:
            dn = (((a.ndim - 1,), (0,)), ((), ()))
        return jax.lax.dot_general(a16, b16, dn, preferred_element_type=f32)

    cell_list = []
    for c in range(_NC):
        cell_list.append(relu(bdot(cs_ref[c], Wc_ref[c]) + bc_ref[c:c + 1, :]))

    ds = ds_ref[...]
    sub = relu(bdot(ds.reshape(G * _ND, 128), Wg_ref[...]) + bg_ref[...])
    drug3 = sub.reshape(G, _ND, 128)
    drug3_16 = drug3.astype(bf16).astype(f32)

    wgt = wgt_ref[...]
    wm = []
    for c in range(_NC):
        t_c = bdot(cell_list[c], wgt,
                   (((1,), (1,)), ((), ())))
        t_c16 = t_c.astype(bf16).astype(f32)
        s_c = jnp.sum(drug3_16 * t_c16[:, None, :], axis=-1)
        gw = jax.nn.sigmoid(s_c)
        wm.append(jnp.where(gw >= 0.5, gw, 0.0))

    degd = jnp.full((G, _ND), 1.0, jnp.float32)
    for c in range(_NC):
        degd = degd + wm[c]
    dinv_d = jax.lax.rsqrt(degd)
    dinv_c = [jax.lax.rsqrt(1.0 + jnp.sum(wm[c], axis=1, keepdims=True))
              for c in range(_NC)]

    def propagate(Xd3, Xc_list, F):
        Yd = dinv_d[:, :, None] * Xd3
        Yc = [dinv_c[c] * Xc_list[c] for c in range(_NC)]
        acc = Yd
        for c in range(_NC):
            acc = acc + wm[c][:, :, None] * Yc[c][:, None, :]
        AXd = dinv_d[:, :, None] * acc
        AXc = []
        for c in range(_NC):
            r = Yc[c] + jnp.sum(wm[c][:, :, None] * Yd, axis=1)
            AXc.append(dinv_c[c] * r)
        return AXd, AXc

    def stack_nodes(Xd3, Xc_list, F):
        Xc3 = jnp.concatenate([x[:, None, :] for x in Xc_list], axis=1)
        return jnp.concatenate([Xd3, Xc3], axis=1).reshape(G * _N, F)

    AXd, AXc = propagate(drug3, cell_list, 128)
    AX = stack_nodes(AXd, AXc, 128)
    H = relu(jnp.dot(AX, W1_ref[...]) + b1_ref[...])
    H3 = H.reshape(G, _N, 256)
    Hd3 = H3[:, :_ND, :]
    Hc_list = [H3[:, _ND + c, :] for c in range(_NC)]

    A2d, A2c = propagate(Hd3, Hc_list, 256)
    A2 = stack_nodes(A2d, A2c, 256)
    mu = jnp.dot(A2, Wmu_ref[...]) + bmu_ref[...]
    lv = jnp.dot(A2, Wlv_ref[...]) + blv_ref[...]
    mu3 = mu.reshape(G, _N, 128)
    mu_ref[...] = mu3
    lv_ref[...] = lv.reshape(G, _N, 128)

    scale = 1.0 / jnp.sqrt(jnp.float32(_DH))
    obar_rows = []
    for g in range(G):
        z_g = mu3[g]
        q = jnp.dot(z_g, Wq_ref[...])
        k = jnp.dot(z_g, Wk_ref[...])
        v = jnp.dot(z_g, Wv_ref[...])
        parts = []
        for h in range(_HEADS):
            sl = slice(h * _DH, (h + 1) * _DH)
            S = jax.lax.dot_general(q[:, sl], k[:, sl],
                                    (((1,), (1,)), ((), ()))) * scale
            S = S - jnp.max(S, axis=-1, keepdims=True)
            E = jnp.exp(S)
            att = E / jnp.sum(E, axis=-1, keepdims=True)
            abar = jnp.sum(att, axis=0, keepdims=True) * (1.0 / _N)
            parts.append(jnp.dot(abar, v[:, sl]))
        obar_rows.append(jnp.concatenate(parts, axis=1))
    obar = jnp.concatenate(obar_rows, axis=0)

    hg = jnp.dot(obar, Wo_ref[...])
    p1 = relu(jnp.dot(hg, Wp1_ref[...]) + bp1_ref[...])
    pred_ref[...] = jax.nn.sigmoid(jnp.dot(p1, Wp2_ref[...]) + bp2_ref[...])


def kernel(cell_subs, drug_subs, batch, drug_cell_batch, W_cell, b_cell,
           W_gnn, b_gnn, weight, W1, b1, W_mu, b_mu, W_lv, b_lv,
           Wq, Wk, Wv, Wo, Wp1, bp1, Wp2, bp2):
    del batch, drug_cell_batch
    ds3 = drug_subs.reshape(_B, _ND, 128)
    grid = (_B // _G,)

    def full(a):
        return pl.BlockSpec(a.shape, lambda i: (0,) * a.ndim)

    in_specs = [
        pl.BlockSpec((_NC, _G, 128), lambda i: (0, i, 0)),
        pl.BlockSpec((_G, _ND, 128), lambda i: (i, 0, 0)),
        full(W_cell), full(b_cell),
        full(W_gnn), pl.BlockSpec((1, 128), lambda i: (0, 0)),
        full(weight),
        full(W1), pl.BlockSpec((1, 256), lambda i: (0, 0)),
        full(W_mu), pl.BlockSpec((1, 128), lambda i: (0, 0)),
        full(W_lv), pl.BlockSpec((1, 128), lambda i: (0, 0)),
        full(Wq), full(Wk), full(Wv), full(Wo),
        full(Wp1), pl.BlockSpec((1, 256), lambda i: (0, 0)),
        full(Wp2), pl.BlockSpec((1, 1), lambda i: (0, 0)),
    ]
    out_specs = [
        pl.BlockSpec((_G, _N, 128), lambda i: (i, 0, 0)),
        pl.BlockSpec((_G, _N, 128), lambda i: (i, 0, 0)),
        pl.BlockSpec((_G, 1), lambda i: (i, 0)),
    ]
    out_shapes = [
        jax.ShapeDtypeStruct((_B, _N, 128), jnp.float32),
        jax.ShapeDtypeStruct((_B, _N, 128), jnp.float32),
        jax.ShapeDtypeStruct((_B, 1), jnp.float32),
    ]
    mu3, lv3, pred = pl.pallas_call(
        _body,
        grid=grid,
        in_specs=in_specs,
        out_specs=out_specs,
        out_shape=out_shapes,
        compiler_params=pltpu.CompilerParams(
            dimension_semantics=("parallel",)),
    )(cell_subs, ds3, W_cell, b_cell, W_gnn, b_gnn.reshape(1, 128), weight,
      W1, b1.reshape(1, 256), W_mu, b_mu.reshape(1, 128),
      W_lv, b_lv.reshape(1, 128), Wq, Wk, Wv, Wo,
      Wp1, bp1.reshape(1, 256), Wp2, bp2.reshape(1, 1))

    mu_flat = mu3.reshape(-1, 128)
    return (pred, mu_flat, lv3.reshape(-1, 128), mu_flat)


# speedup vs baseline: 3.0504x; 2.4370x over previous
import jax
import jax.numpy as jnp
from jax.experimental import pallas as pl
from jax.experimental.pallas import tpu as pltpu

_B = 256
_ND = 64
_NC = 8
_N = _ND + _NC
_G = 8
_HEADS = 4
_DH = 32


def _body(cs_ref, ds_ref, Wc_ref, bc_ref, Wg_ref, bg_ref, wgt_ref,
          W1_ref, b1_ref, Wmu_ref, bmu_ref, Wlv_ref, blv_ref,
          Wq_ref, Wk_ref, Wv_ref, Wo_ref, Wp1_ref, bp1_ref, Wp2_ref, bp2_ref,
          mu_ref, lv_ref, pred_ref):
    G = _G
    relu = lambda x: jnp.maximum(x, 0.0)
    bf16 = jnp.bfloat16
    f32 = jnp.float32

    def bdot(a, b, dn=None):
        a16, b16 = a.astype(bf16), b.astype(bf16)
        if dn is None:
            dn = (((a.ndim - 1,), (0,)), ((), ()))
        return jax.lax.dot_general(a16, b16, dn, preferred_element_type=f32)

    cell_list = []
    for c in range(_NC):
        cell_list.append(relu(bdot(cs_ref[c], Wc_ref[c]) + bc_ref[c:c + 1, :]))
    cell3 = jnp.concatenate([x[:, None, :] for x in cell_list], axis=1)

    ds = ds_ref[...]
    sub = relu(bdot(ds.reshape(G * _ND, 128), Wg_ref[...]) + bg_ref[...])
    drug3 = sub.reshape(G, _ND, 128)

    T3 = jax.lax.dot_general(cell3.astype(bf16), wgt_ref[...].astype(bf16),
                             (((2,), (1,)), ((), ())),
                             preferred_element_type=f32)
    s3 = jax.lax.dot_general(drug3.astype(bf16), T3.astype(bf16),
                             (((2,), (2,)), ((0,), (0,))),
                             preferred_element_type=f32)
    gw = jax.nn.sigmoid(s3)
    wm3 = jnp.where(gw >= 0.5, gw, 0.0)

    dinv_d = jax.lax.rsqrt(1.0 + jnp.sum(wm3, axis=2))[:, :, None]
    dinv_c = jax.lax.rsqrt(1.0 + jnp.sum(wm3, axis=1))[:, :, None]

    def propagate(Xd3, Xc3):
        Yd = dinv_d * Xd3
        Yc = dinv_c * Xc3
        AXd = dinv_d * (Yd + jax.lax.dot_general(
            wm3, Yc, (((2,), (1,)), ((0,), (0,))), preferred_element_type=f32))
        AXc = dinv_c * (Yc + jax.lax.dot_general(
            wm3, Yd, (((1,), (1,)), ((0,), (0,))), preferred_element_type=f32))
        return jnp.concatenate([AXd, AXc], axis=1)

    AX = propagate(drug3, cell3).reshape(G * _N, 128)
    H = relu(jnp.dot(AX, W1_ref[...]) + b1_ref[...])
    H3 = H.reshape(G, _N, 256)

    A2 = propagate(H3[:, :_ND, :], H3[:, _ND:, :]).reshape(G * _N, 256)
    mu = jnp.dot(A2, Wmu_ref[...]) + bmu_ref[...]
    lv = jnp.dot(A2, Wlv_ref[...]) + blv_ref[...]
    mu_ref[...] = mu.reshape(G, _N, 128)
    lv_ref[...] = lv.reshape(G, _N, 128)

    scale = 1.0 / jnp.sqrt(jnp.float32(_DH))
    q3 = jnp.dot(mu, Wq_ref[...]).reshape(G, _N, 128)
    k3 = jnp.dot(mu, Wk_ref[...]).reshape(G, _N, 128)
    v3 = jnp.dot(mu, Wv_ref[...]).reshape(G, _N, 128)
    parts = []
    for h in range(_HEADS):
        sl = slice(h * _DH, (h + 1) * _DH)
        S = jax.lax.dot_general(q3[:, :, sl], k3[:, :, sl],
                                (((2,), (2,)), ((0,), (0,))),
                                preferred_element_type=f32) * scale
        S = S - jnp.max(S, axis=-1, keepdims=True)
        E = jnp.exp(S)
        att = E / jnp.sum(E, axis=-1, keepdims=True)
        abar = jnp.sum(att, axis=1) * (1.0 / _N)
        parts.append(jax.lax.dot_general(abar, v3[:, :, sl],
                                         (((1,), (1,)), ((0,), (0,))),
                                         preferred_element_type=f32))
    obar = jnp.concatenate(parts, axis=-1)

    hg = jnp.dot(obar, Wo_ref[...])
    p1 = relu(jnp.dot(hg, Wp1_ref[...]) + bp1_ref[...])
    pred_ref[...] = jax.nn.sigmoid(jnp.dot(p1, Wp2_ref[...]) + bp2_ref[...])


def kernel(cell_subs, drug_subs, batch, drug_cell_batch, W_cell, b_cell,
           W_gnn, b_gnn, weight, W1, b1, W_mu, b_mu, W_lv, b_lv,
           Wq, Wk, Wv, Wo, Wp1, bp1, Wp2, bp2):
    del batch, drug_cell_batch
    ds3 = drug_subs.reshape(_B, _ND, 128)
    grid = (_B // _G,)

    def full(a):
        return pl.BlockSpec(a.shape, lambda i: (0,) * a.ndim)

    in_specs = [
        pl.BlockSpec((_NC, _G, 128), lambda i: (0, i, 0)),
        pl.BlockSpec((_G, _ND, 128), lambda i: (i, 0, 0)),
        full(W_cell), full(b_cell),
        full(W_gnn), pl.BlockSpec((1, 128), lambda i: (0, 0)),
        full(weight),
        full(W1), pl.BlockSpec((1, 256), lambda i: (0, 0)),
        full(W_mu), pl.BlockSpec((1, 128), lambda i: (0, 0)),
        full(W_lv), pl.BlockSpec((1, 128), lambda i: (0, 0)),
        full(Wq), full(Wk), full(Wv), full(Wo),
        full(Wp1), pl.BlockSpec((1, 256), lambda i: (0, 0)),
        full(Wp2), pl.BlockSpec((1, 1), lambda i: (0, 0)),
    ]
    out_specs = [
        pl.BlockSpec((_G, _N, 128), lambda i: (i, 0, 0)),
        pl.BlockSpec((_G, _N, 128), lambda i: (i, 0, 0)),
        pl.BlockSpec((_G, 1), lambda i: (i, 0)),
    ]
    out_shapes = [
        jax.ShapeDtypeStruct((_B, _N, 128), jnp.float32),
        jax.ShapeDtypeStruct((_B, _N, 128), jnp.float32),
        jax.ShapeDtypeStruct((_B, 1), jnp.float32),
    ]
    mu3, lv3, pred = pl.pallas_call(
        _body,
        grid=grid,
        in_specs=in_specs,
        out_specs=out_specs,
        out_shape=out_shapes,
        compiler_params=pltpu.CompilerParams(
            dimension_semantics=("parallel",)),
    )(cell_subs, ds3, W_cell, b_cell, W_gnn, b_gnn.reshape(1, 128), weight,
      W1, b1.reshape(1, 256), W_mu, b_mu.reshape(1, 128),
      W_lv, b_lv.reshape(1, 128), Wq, Wk, Wv, Wo,
      Wp1, bp1.reshape(1, 256), Wp2, bp2.reshape(1, 1))

    mu_flat = mu3.reshape(-1, 128)
    return (pred, mu_flat, lv3.reshape(-1, 128), mu_flat)
